# Initial kernel scaffold; baseline (speedup 1.0000x reference)
import jax
import jax.numpy as jnp
from jax.experimental import pallas as pl


def kernel(node_feat, pos, edge_index, W_in, b_in, W_m1, b_m1, W_m2, b_m2, W_u1, b_u1, W_u2, b_u2, ln_g, ln_b):
    raise NotImplementedError("write your pallas kernel here")


# speedup vs baseline: 3.1737x; 3.1737x over previous
import functools

import jax
import jax.numpy as jnp
from jax import lax
from jax.experimental import pallas as pl
from jax.experimental.pallas import tpu as pltpu
from jax.experimental.pallas import tpu_sc as plsc

F32 = jnp.float32
NC, NS, LANES = 2, 16, 16
NW = NC * NS


def _nodes_pre(node_feat, pos, W_in, b_in, W_m1a, b_m1):
    n, d = node_feat.shape
    h_dim = W_in.shape[1]
    bn = 1000

    def body(nf, ps, win, bi, wm1a, bm1, h_o, h1_o, tab_o):
        h = jnp.dot(nf[...], win[...], preferred_element_type=F32) + bi[...]
        h1 = jnp.dot(h, wm1a[...], preferred_element_type=F32) + bm1[...]
        s = jnp.sum(h, axis=1, keepdims=True)
        h_o[...] = h
        h1_o[...] = h1
        tab_o[...] = jnp.concatenate(
            [ps[...], s, jnp.zeros((ps.shape[0], 12), F32)], axis=1)

    return pl.pallas_call(
        body,
        grid=(n // bn,),
        in_specs=[
            pl.BlockSpec((bn, d), lambda i: (i, 0)),
            pl.BlockSpec((bn, 3), lambda i: (i, 0)),
            pl.BlockSpec((d, h_dim), lambda i: (0, 0)),
            pl.BlockSpec((1, h_dim), lambda i: (0, 0)),
            pl.BlockSpec((h_dim, h_dim), lambda i: (0, 0)),
            pl.BlockSpec((1, h_dim), lambda i: (0, 0)),
        ],
        out_specs=[
            pl.BlockSpec((bn, h_dim), lambda i: (i, 0)),
            pl.BlockSpec((bn, h_dim), lambda i: (i, 0)),
            pl.BlockSpec((bn, 16), lambda i: (i, 0)),
        ],
        out_shape=[
            jax.ShapeDtypeStruct((n, h_dim), F32),
            jax.ShapeDtypeStruct((n, h_dim), F32),
            jax.ShapeDtypeStruct((n, 16), F32),
        ],
    )(node_feat, pos, W_in, b_in, W_m1a, b_m1)


def _edge_gather(src, dst, h1, tab):
    e = src.shape[0]
    h_dim = h1.shape[1]
    epw = e // NW
    c = 80
    nch = epw // c
    mesh = plsc.VectorSubcoreMesh(core_axis_name="c", subcore_axis_name="s")

    @functools.partial(
        pl.kernel,
        out_type=(
            jax.ShapeDtypeStruct((e, h_dim), F32),
            jax.ShapeDtypeStruct((e, 16), F32),
            jax.ShapeDtypeStruct((e, 16), F32),
        ),
        mesh=mesh,
        scratch_types=[
            pltpu.VMEM((c,), jnp.int32),
            pltpu.VMEM((c,), jnp.int32),
            pltpu.VMEM((c, h_dim), F32),
            pltpu.VMEM((c, 16), F32),
            pltpu.VMEM((c, 16), F32),
            pltpu.SemaphoreType.DMA,
            pltpu.SemaphoreType.DMA,
            pltpu.SemaphoreType.DMA,
        ],
        compiler_params=pltpu.CompilerParams(use_tc_tiling_on_sc=False),
    )
    def k2(src_h, dst_h, h1_h, tab_h, oh1, ots, otd,
           idxs, idxd, rows, tabs, tabd, s1, s2, s3):
        wid = lax.axis_index("s") * NC + lax.axis_index("c")

        def chunk(i, carry):
            base = pl.multiple_of(wid * epw + i * c, 8)
            pltpu.sync_copy(src_h.at[pl.ds(base, c)], idxs)
            pltpu.sync_copy(dst_h.at[pl.ds(base, c)], idxd)
            cp1 = pltpu.async_copy(h1_h.at[idxs], rows, s1)
            cp2 = pltpu.async_copy(tab_h.at[idxs], tabs, s2)
            cp3 = pltpu.async_copy(tab_h.at[idxd], tabd, s3)
            cp1.wait()
            cp2.wait()
            cp3.wait()
            pltpu.sync_copy(rows, oh1.at[pl.ds(base, c)])
            pltpu.sync_copy(tabs, ots.at[pl.ds(base, c)])
            pltpu.sync_copy(tabd, otd.at[pl.ds(base, c)])
            return carry

        lax.fori_loop(0, nch, chunk, 0)

    return k2(src, dst, h1, tab)


def _edge_math(h1src, tabs, tabd, W_rbf, cut, n_rbf):
    e, h_dim = h1src.shape
    be = 2000

    def body(hs, ts, td, wr, u_o, v_o):
        rij = td[:, 0:3] - ts[:, 0:3]
        d2 = jnp.sum(rij * rij, axis=1, keepdims=True)
        dij = jnp.sqrt(d2)
        dirn = rij / (dij + 1e-8)
        gate = jax.nn.sigmoid(ts[:, 3:4])
        centers = (lax.broadcasted_iota(jnp.int32, (1, n_rbf), 1)
                   .astype(F32) * (cut / (n_rbf - 1)))
        gamma = 1.0 / (2.0 * (cut / n_rbf) ** 2)
        rbf = jnp.exp(-gamma * (dij - centers) ** 2)
        t = hs[...] + jnp.dot(rbf, wr[...], preferred_element_type=F32)
        u_o[...] = t * jax.nn.sigmoid(t)
        v_o[...] = jnp.concatenate(
            [gate * dirn, jnp.ones((be, 1), F32), jnp.zeros((be, 12), F32)],
            axis=1)

    return pl.pallas_call(
        body,
        grid=(e // be,),
        in_specs=[
            pl.BlockSpec((be, h_dim), lambda i: (i, 0)),
            pl.BlockSpec((be, 16), lambda i: (i, 0)),
            pl.BlockSpec((be, 16), lambda i: (i, 0)),
            pl.BlockSpec((n_rbf, h_dim), lambda i: (0, 0)),
        ],
        out_specs=[
            pl.BlockSpec((be, h_dim), lambda i: (i, 0)),
            pl.BlockSpec((be, 16), lambda i: (i, 0)),
        ],
        out_shape=[
            jax.ShapeDtypeStruct((e, h_dim), F32),
            jax.ShapeDtypeStruct((e, 16), F32),
        ],
    )(h1src, tabs, tabd, W_rbf)


def _edge_scatter(dst, u, vrow, n):
    e, h_dim = u.shape
    epw = e // NW
    c = 80
    nch = epw // c
    rpt = n // NS
    dn = 125
    mesh = plsc.VectorSubcoreMesh(core_axis_name="c", subcore_axis_name="s")

    @functools.partial(
        pl.kernel,
        out_type=(
            jax.ShapeDtypeStruct((NC, n, h_dim), F32),
            jax.ShapeDtypeStruct((NC, n, 16), F32),
        ),
        mesh=mesh,
        scratch_types=[
            pltpu.VMEM((c,), jnp.int32),
            pltpu.VMEM((max(c, dn), h_dim), F32),
            pltpu.VMEM((max(c, dn), 16), F32),
            pltpu.VMEM_SHARED((n, h_dim), F32),
            pltpu.VMEM_SHARED((n, 16), F32),
            pltpu.SemaphoreType.DMA,
            pltpu.SemaphoreType.DMA,
        ],
        compiler_params=pltpu.CompilerParams(use_tc_tiling_on_sc=False),
    )
    def k4(dst_h, u_h, v_h, oagg, ovn,
           idxd, ubuf, vbuf, agg_sh, vn_sh, s1, s2):
        cid = lax.axis_index("c")
        sid = lax.axis_index("s")
        wid = sid * NC + cid
        r0 = sid * rpt

        ubuf[...] = jnp.zeros_like(ubuf)
        vbuf[...] = jnp.zeros_like(vbuf)

        def zloop(j, carry):
            pltpu.sync_copy(ubuf.at[pl.ds(0, dn)],
                            agg_sh.at[pl.ds(r0 + j * dn, dn)])
            pltpu.sync_copy(vbuf.at[pl.ds(0, dn)],
                            vn_sh.at[pl.ds(r0 + j * dn, dn)])
            return carry

        lax.fori_loop(0, rpt // dn, zloop, 0)
        plsc.subcore_barrier()

        def chunk(i, carry):
            base = pl.multiple_of(wid * epw + i * c, 8)
            pltpu.sync_copy(dst_h.at[pl.ds(base, c)], idxd)
            pltpu.sync_copy(u_h.at[pl.ds(base, c)], ubuf.at[pl.ds(0, c)])
            pltpu.sync_copy(v_h.at[pl.ds(base, c)], vbuf.at[pl.ds(0, c)])
            cp1 = pltpu.async_copy(ubuf.at[pl.ds(0, c)], agg_sh.at[idxd],
                                   s1, add=True)
            cp2 = pltpu.async_copy(vbuf.at[pl.ds(0, c)], vn_sh.at[idxd],
                                   s2, add=True)
            cp1.wait()
            cp2.wait()
            return carry

        lax.fori_loop(0, nch, chunk, 0)
        plsc.subcore_barrier()

        def dump(j, carry):
            rr = r0 + j * dn
            pltpu.sync_copy(agg_sh.at[pl.ds(rr, dn)], ubuf.at[pl.ds(0, dn)])
            pltpu.sync_copy(ubuf.at[pl.ds(0, dn)], oagg.at[cid, pl.ds(rr, dn)])
            pltpu.sync_copy(vn_sh.at[pl.ds(rr, dn)], vbuf.at[pl.ds(0, dn)])
            pltpu.sync_copy(vbuf.at[pl.ds(0, dn)], ovn.at[cid, pl.ds(rr, dn)])
            return carry

        lax.fori_loop(0, rpt // dn, dump, 0)

    return k4(dst, u, vrow)


def _node_update(h, aggu, vn, W_m2, b_m2, W_u1a, W_u1b, b_u1, W_u2, b_u2,
                 ln_g, ln_b):
    n, h_dim = h.shape
    bn = 1000
    nblk = n // bn

    def body(hb, ab, vb, wm2, bm2, wu1a, wu1b, bu1, wu2, bu2, g, b,
             h2_o, gp_o, gv_o):
        i = pl.program_id(0)
        au = ab[0] + ab[1]
        deg = vb[0, :, 3:4] + vb[1, :, 3:4]
        agg = jnp.dot(au, wm2[...], preferred_element_type=F32) + deg * bm2[...]
        pre = (jnp.dot(hb[...], wu1a[...], preferred_element_type=F32)
               + jnp.dot(agg, wu1b[...], preferred_element_type=F32)
               + bu1[...])
        x = pre * jax.nn.sigmoid(pre)
        x = jnp.dot(x, wu2[...], preferred_element_type=F32) + bu2[...]
        mu = jnp.mean(x, axis=1, keepdims=True)
        var = jnp.mean((x - mu) ** 2, axis=1, keepdims=True)
        h2 = (x - mu) * jax.lax.rsqrt(var + 1e-5) * g[...] + b[...]
        h2_o[...] = h2

        @pl.when(i == 0)
        def _init():
            gp_o[...] = jnp.zeros_like(gp_o)
            gv_o[...] = jnp.zeros_like(gv_o)

        gp_o[...] += jnp.sum(h2, axis=0, keepdims=True)
        vpart = jnp.sum(vb[...], axis=(0, 1))
        gv_o[...] += jnp.concatenate(
            [vpart[None, :], jnp.zeros((1, 112), F32)], axis=1)

        @pl.when(i == nblk - 1)
        def _fin():
            gp_o[...] = gp_o[...] / float(n)
            v = gv_o[...] / float(n)
            lane = lax.broadcasted_iota(jnp.int32, (1, 128), 1)
            vm = jnp.where(lane < 3, v, 0.0)
            nrm = jnp.sqrt(jnp.sum(vm * vm)) + 1e-8
            gv_o[...] = vm / nrm

    return pl.pallas_call(
        body,
        grid=(nblk,),
        in_specs=[
            pl.BlockSpec((bn, h_dim), lambda i: (i, 0)),
            pl.BlockSpec((NC, bn, h_dim), lambda i: (0, i, 0)),
            pl.BlockSpec((NC, bn, 16), lambda i: (0, i, 0)),
            pl.BlockSpec((h_dim, h_dim), lambda i: (0, 0)),
            pl.BlockSpec((1, h_dim), lambda i: (0, 0)),
            pl.BlockSpec((h_dim, h_dim), lambda i: (0, 0)),
            pl.BlockSpec((h_dim, h_dim), lambda i: (0, 0)),
            pl.BlockSpec((1, h_dim), lambda i: (0, 0)),
            pl.BlockSpec((h_dim, h_dim), lambda i: (0, 0)),
            pl.BlockSpec((1, h_dim), lambda i: (0, 0)),
            pl.BlockSpec((1, h_dim), lambda i: (0, 0)),
            pl.BlockSpec((1, h_dim), lambda i: (0, 0)),
        ],
        out_specs=[
            pl.BlockSpec((bn, h_dim), lambda i: (i, 0)),
            pl.BlockSpec((1, h_dim), lambda i: (0, 0)),
            pl.BlockSpec((1, 128), lambda i: (0, 0)),
        ],
        out_shape=[
            jax.ShapeDtypeStruct((n, h_dim), F32),
            jax.ShapeDtypeStruct((1, h_dim), F32),
            jax.ShapeDtypeStruct((1, 128), F32),
        ],
    )(h, aggu, vn, W_m2, b_m2, W_u1a, W_u1b, b_u1, W_u2, b_u2, ln_g, ln_b)


def kernel(node_feat, pos, edge_index, W_in, b_in, W_m1, b_m1, W_m2, b_m2,
           W_u1, b_u1, W_u2, b_u2, ln_g, ln_b):
    n, d = node_feat.shape
    h_dim = W_in.shape[1]
    n_rbf = W_m1.shape[0] - h_dim
    cut = 8.0

    src = edge_index[0]
    dst = edge_index[1]
    W_m1a = W_m1[:h_dim]
    W_rbf = W_m1[h_dim:]
    W_u1a = W_u1[:h_dim]
    W_u1b = W_u1[h_dim:]

    h, h1, tab = _nodes_pre(node_feat, pos, W_in, b_in[None], W_m1a,
                            b_m1[None])
    h1src, tabs, tabd = _edge_gather(src, dst, h1, tab)
    u, vrow = _edge_math(h1src, tabs, tabd, W_rbf, cut, n_rbf)
    aggu, vn = _edge_scatter(dst, u, vrow, n)
    h2, gp, gv = _node_update(h, aggu, vn, W_m2, b_m2[None], W_u1a, W_u1b,
                              b_u1[None], W_u2, b_u2[None], ln_g[None],
                              ln_b[None])
    return (h2, gp[0], gv[0, 0:3])
